# baseline (speedup 1.0000x reference)
import functools

import jax
import jax.numpy as jnp
from jax import lax
from jax.experimental import pallas as pl
from jax.experimental.pallas import tpu as pltpu
from jax.experimental.pallas import tpu_sc as plsc

F32 = jnp.float32

NC = 2
NS = 16
NW = NC * NS
RB = 16
BM = 512


@functools.lru_cache(maxsize=None)
def _make_spmm(R, C, second):
    rpw = R // NW
    rb = RB if C <= 256 else RB // 2
    if (rpw // rb) % 2:
        rb //= 2
    nsteps = rpw // rb
    Cv = C // 16
    mesh = plsc.VectorSubcoreMesh(
        core_axis_name="c", subcore_axis_name="s",
        num_cores=NC, num_subcores=NS)

    scratch = [
        pltpu.VMEM((rpw * 8,), jnp.int32),
        pltpu.VMEM((rpw * 8,), F32),
        pltpu.VMEM((2, rb * 8, C), F32),
        pltpu.VMEM((2, rb, C), F32),
        pltpu.VMEM((2, rb, C), F32),
        pltpu.SemaphoreType.DMA((2,)),
        pltpu.SemaphoreType.DMA((2,)),
        pltpu.SemaphoreType.DMA((2,)),
    ]

    @functools.partial(
        pl.kernel,
        out_type=jax.ShapeDtypeStruct((R, C), F32),
        mesh=mesh,
        scratch_types=scratch,
    )
    def spmm(x_hbm, idx_hbm, w_hbm, *rest):
        if second:
            x0_hbm, out_hbm = rest[0], rest[1]
        else:
            out_hbm = rest[0]
            x0_hbm = None
        idx_v, w_v, rows, outb, x0b, sg, so, sx = rest[-8:]
        wid = lax.axis_index("s") * NC + lax.axis_index("c")
        base = wid * rpw
        pltpu.sync_copy(idx_hbm.at[pl.ds(base * 8, rpw * 8)], idx_v)
        pltpu.sync_copy(w_hbm.at[pl.ds(base * 8, rpw * 8)], w_v)

        def issue(s, p):
            pltpu.async_copy(
                x_hbm.at[idx_v.at[pl.ds(s * (rb * 8), rb * 8)]],
                rows.at[p], sg.at[p])
            if second:
                pltpu.async_copy(
                    x0_hbm.at[pl.ds(base + s * rb, rb)], x0b.at[p], sx.at[p])

        def compute(s, p):
            pltpu.make_async_copy(
                x_hbm.at[idx_v.at[pl.ds(s * (rb * 8), rb * 8)]],
                rows.at[p], sg.at[p]).wait()
            if second:
                pltpu.make_async_copy(
                    x0_hbm.at[pl.ds(base, rb)], x0b.at[p], sx.at[p]).wait()

            def rowpair(rr, carry2):
                wv = w_v[pl.ds(s * (rb * 8) + rr * 16, 16)]
                for half in range(2):
                    r = rr * 2 + half
                    accs = [jnp.zeros((16,), F32)] * Cv
                    for j in range(8):
                        lane = jnp.full((16,), half * 8 + j, jnp.int32)
                        wsp = wv.at[lane].get(mode="promise_in_bounds")
                        for c in range(Cv):
                            accs[c] = accs[c] + wsp * rows[
                                p, r * 8 + j, pl.ds(c * 16, 16)]
                    for c in range(Cv):
                        if second:
                            outb[p, r, pl.ds(c * 16, 16)] = (
                                2.0 * accs[c] - x0b[p, r, pl.ds(c * 16, 16)])
                        else:
                            outb[p, r, pl.ds(c * 16, 16)] = accs[c]
                return carry2

            lax.fori_loop(0, rb // 2, rowpair, 0, unroll=False)
            pltpu.async_copy(
                outb.at[p], out_hbm.at[pl.ds(base + s * rb, rb)], so.at[p])

        def drain_out(p):
            pltpu.make_async_copy(
                outb.at[p], out_hbm.at[pl.ds(base, rb)], so.at[p]).wait()

        issue(0, 0)
        issue(1, 1)

        def k_iter(k, carry):
            s0 = 2 * k

            @pl.when(k > 0)
            def _():
                drain_out(0)
            compute(s0, 0)

            @pl.when(s0 + 2 < nsteps)
            def _():
                issue(s0 + 2, 0)

            @pl.when(k > 0)
            def _():
                drain_out(1)
            compute(s0 + 1, 1)

            @pl.when(s0 + 3 < nsteps)
            def _():
                issue(s0 + 3, 1)
            return carry

        lax.fori_loop(0, nsteps // 2, k_iter, 0, unroll=False)
        drain_out(0)
        drain_out(1)

    return spmm


def _spmm(xf, idx, w):
    R, C = xf.shape
    return _make_spmm(R, C, False)(xf, idx, w)


def _spmm2(xf, idx, w, x0):
    R, C = xf.shape
    return _make_spmm(R, C, True)(xf, idx, w, x0)


@functools.lru_cache(maxsize=None)
def _make_combine(M, cins, cout, with_bias, with_stats, with_acc=False):
    n = len(cins)
    grid = (M // BM,)
    in_specs = [pl.BlockSpec((BM, cin), lambda i: (i, 0)) for cin in cins]
    in_specs += [pl.BlockSpec((cin, cout), lambda i: (0, 0)) for cin in cins]
    if with_acc:
        in_specs.append(pl.BlockSpec((BM, cout), lambda i: (i, 0)))
    if with_bias:
        in_specs.append(pl.BlockSpec((8, cout), lambda i: (0, 0)))
    out_shape = [jax.ShapeDtypeStruct((M, cout), F32)]
    out_specs = [pl.BlockSpec((BM, cout), lambda i: (i, 0))]
    if with_stats:
        out_shape.append(jax.ShapeDtypeStruct((8, cout), F32))
        out_specs.append(pl.BlockSpec((8, cout), lambda i: (0, 0)))

    def body(*refs):
        xr = refs[:n]
        wr = refs[n:2 * n]
        k = 2 * n
        ar = refs[k] if with_acc else None
        k += 1 if with_acc else 0
        br = refs[k] if with_bias else None
        k += 1 if with_bias else 0
        out_ref = refs[k]
        st_ref = refs[k + 1] if with_stats else None

        acc = jnp.dot(xr[0][...], wr[0][...], preferred_element_type=F32)
        for t in range(1, n):
            acc = acc + jnp.dot(xr[t][...], wr[t][...],
                                preferred_element_type=F32)
        if with_acc:
            acc = ar[...] + acc
        if with_bias:
            acc = acc + br[0, :][None, :]
        out_ref[...] = acc
        if with_stats:
            @pl.when(pl.program_id(0) == 0)
            def _():
                st_ref[...] = jnp.zeros_like(st_ref)
            st_ref[0, :] += jnp.sum(acc, axis=0)
            st_ref[1, :] += jnp.sum(acc * acc, axis=0)

    return pl.pallas_call(body, grid=grid, in_specs=in_specs,
                          out_specs=out_specs, out_shape=out_shape)


def _combine(xs, ws, bias=None, stats=False, acc=None):
    M = xs[0].shape[0]
    cins = tuple(x.shape[1] for x in xs)
    cout = ws[0].shape[1]
    args = list(xs) + list(ws)
    if acc is not None:
        args.append(acc)
    if bias is not None:
        args.append(bias)
    out = _make_combine(M, cins, cout, bias is not None, stats,
                        acc is not None)(*args)
    return out if stats else out[0]


@functools.lru_cache(maxsize=None)
def _make_bnrelu(M, C, with_z, Cp):
    grid = (M // BM,)
    in_specs = [pl.BlockSpec((BM, C), lambda i: (i, 0)),
                pl.BlockSpec((8, C), lambda i: (0, 0))]
    if with_z:
        in_specs.append(pl.BlockSpec((BM, C), lambda i: (i, 0)))

    def body(*refs):
        o_ref, p_ref = refs[0], refs[1]
        y_ref = refs[-1]
        y = jnp.maximum(o_ref[...] * p_ref[0, :][None, :]
                        + p_ref[1, :][None, :], 0.0)
        if with_z:
            y = y + refs[2][...] + p_ref[2, :][None, :]
        if Cp > C:
            y = jnp.concatenate([y, jnp.zeros((BM, Cp - C), F32)], axis=1)
        y_ref[...] = y

    return pl.pallas_call(
        body, grid=grid, in_specs=in_specs,
        out_specs=pl.BlockSpec((BM, Cp), lambda i: (i, 0)),
        out_shape=jax.ShapeDtypeStruct((M, Cp), F32))


def _bn_scale_shift(st, g, be, rows, resbias=None):
    m = st[0] / rows
    v = st[1] / rows - m * m
    scale = g * lax.rsqrt(v + 1e-5)
    shift = be - m * scale
    p = jnp.zeros((8, st.shape[1]), F32).at[0].set(scale).at[1].set(shift)
    if resbias is not None:
        p = p.at[2].set(resbias)
    return p


def _bnrelu(out, p, z=None, cpad=None):
    M, C = out.shape
    Cp = C if cpad is None else cpad
    if z is None:
        return _make_bnrelu(M, C, False, Cp)(out, p)
    return _make_bnrelu(M, C, True, Cp)(out, p, z)


@functools.lru_cache(maxsize=None)
def _make_pool(G, C):
    bg = min(BM, G)
    grid = (G // bg,)
    in_specs = [pl.BlockSpec((bg, C), lambda i, j=j: (i, j)) for j in range(4)]

    def body(a0, a1, a2, a3, m_ref, i_ref):
        x0, x1, x2, x3 = a0[...], a1[...], a2[...], a3[...]
        m = jnp.maximum(jnp.maximum(x0, x1), jnp.maximum(x2, x3))
        m_ref[...] = m
        i_ref[...] = jnp.where(
            x0 == m, 0,
            jnp.where(x1 == m, 1, jnp.where(x2 == m, 2, 3))).astype(jnp.int32)

    return pl.pallas_call(
        body, grid=grid, in_specs=in_specs,
        out_specs=[pl.BlockSpec((bg, C), lambda i: (i, 0)),
                   pl.BlockSpec((bg, C), lambda i: (i, 0))],
        out_shape=[jax.ShapeDtypeStruct((G, C), F32),
                   jax.ShapeDtypeStruct((G, C), jnp.int32)])


def _pool(xf):
    R, C = xf.shape
    xg = xf.reshape(R // 4, 4 * C)
    return _make_pool(R // 4, C)(xg, xg, xg, xg)


@functools.lru_cache(maxsize=None)
def _make_unpool(G, C):
    bg = min(BM, G)
    grid = (G // bg,)

    def body(x_ref, i_ref, o_ref):
        x = x_ref[...]
        idx = i_ref[...]
        o_ref[...] = jnp.concatenate(
            [jnp.where(idx == j, x, 0.0) for j in range(4)], axis=1)

    return pl.pallas_call(
        body, grid=grid,
        in_specs=[pl.BlockSpec((bg, C), lambda i: (i, 0)),
                  pl.BlockSpec((bg, C), lambda i: (i, 0))],
        out_specs=pl.BlockSpec((bg, 4 * C), lambda i: (i, 0)),
        out_shape=jax.ShapeDtypeStruct((G, 4 * C), F32))


def _unpool(xf, idx):
    G, C = xf.shape
    return _make_unpool(G, C)(xf, idx).reshape(G * 4, C)


def _cheb_split(xf, idx, w, W):
    x1 = _spmm(xf, idx, w)
    c01 = _combine([xf, x1], [W[0], W[1]])
    x2 = _spmm2(x1, idx, w, xf)
    return x2, W[2], c01


def _block(xf, idx, w, W, g, be, z=None, resbias=None, cpad=None):
    x2, W2, c01 = _cheb_split(xf, idx, w, W)
    out, st = _combine([x2], [W2], stats=True, acc=c01)
    p = _bn_scale_shift(st, g, be, out.shape[0], resbias)
    return _bnrelu(out, p, z, cpad)


def _pad_rows(W, rows):
    pad = [(0, 0)] * (W.ndim - 2) + [(0, rows - W.shape[-2]), (0, 0)]
    return jnp.pad(W, pad)


def kernel(x, src0, dst0, lw0, src1, dst1, lw1, src2, dst2, lw2,
           w_conv11, g_conv11, be_conv11, w_conv13, g_conv13, be_conv13,
           w_conv21, g_conv21, be_conv21, w_conv23, g_conv23, be_conv23,
           w_conv31, g_conv31, be_conv31, w_conv33, g_conv33, be_conv33,
           w_uconv21, g_uconv21, be_uconv21, w_uconv22, g_uconv22, be_uconv22,
           w_uconv11, g_uconv11, be_uconv11, w_uconv12, g_uconv12, be_uconv12,
           w_uconv13, b_uconv13,
           w_conv1res, b_conv1res, w_conv2res, b_conv2res,
           w_conv3res, b_conv3res):
    B, N0, C0 = x.shape
    edges = []
    for srcl, lwl, n in ((src0, lw0, N0), (src1, lw1, N0 // 4),
                         (src2, lw2, N0 // 16)):
        srcl = srcl.astype(jnp.int32)
        idx = jnp.concatenate([srcl, srcl + n])
        edges.append((idx, jnp.concatenate([lwl, lwl])))
    (i0, e0), (i1, e1), (i2, e2) = edges

    xf = jnp.pad(x.reshape(B * N0, C0), ((0, 0), (0, 128 - C0)))
    x11 = _block(xf, i0, e0, _pad_rows(w_conv11, 128),
                 g_conv11, be_conv11, cpad=128)
    res1 = _combine([xf], [_pad_rows(w_conv1res, 128)])
    x1 = _block(x11, i0, e0, _pad_rows(w_conv13, 128), g_conv13, be_conv13,
                z=res1, resbias=b_conv1res)
    p1, ix1 = _pool(x1)
    x2 = _block(p1, i1, e1, w_conv21, g_conv21, be_conv21, cpad=256)
    res2 = _combine([p1], [w_conv2res])
    x2 = _block(x2, i1, e1, _pad_rows(w_conv23, 256), g_conv23, be_conv23,
                z=res2, resbias=b_conv2res)
    p2, ix2 = _pool(x2)
    x3 = _block(p2, i2, e2, w_conv31, g_conv31, be_conv31)
    res3 = _combine([p2], [w_conv3res])
    x3 = _block(x3, i2, e2, w_conv33, g_conv33, be_conv33,
                z=res3, resbias=b_conv3res)
    u = _unpool(x3, ix2)
    u = jnp.concatenate([u, x2], axis=1)
    u = _block(u, i1, e1, w_uconv21, g_uconv21, be_uconv21)
    u = _block(u, i1, e1, w_uconv22, g_uconv22, be_uconv22)
    u = _unpool(u, ix1)
    u = jnp.concatenate([u, x1], axis=1)
    u = _block(u, i0, e0, w_uconv11, g_uconv11, be_uconv11)
    u = _block(u, i0, e0, w_uconv12, g_uconv12, be_uconv12, cpad=128)
    u = jnp.concatenate([u, x11], axis=1)
    cout = w_uconv13.shape[2]
    w13 = jnp.zeros((3, 256, cout), F32)
    w13 = w13.at[:, 0:64, :].set(w_uconv13[:, 0:64, :])
    w13 = w13.at[:, 128:192, :].set(w_uconv13[:, 64:128, :])
    x2, W2, c01 = _cheb_split(u, i0, e0, w13)
    bias = jnp.zeros((8, cout), F32).at[0].set(b_uconv13)
    out = _combine([x2], [W2], bias=bias, acc=c01)
    return out.reshape(B, N0, cout)


# speedup vs baseline: 1.1623x; 1.1623x over previous
import functools

import jax
import jax.numpy as jnp
from jax import lax
from jax.experimental import pallas as pl
from jax.experimental.pallas import tpu as pltpu
from jax.experimental.pallas import tpu_sc as plsc

F32 = jnp.float32

NC = 2
NS = 16
NW = NC * NS
RB = 16
BM = 512


@functools.lru_cache(maxsize=None)
def _make_spmm(R, C, second):
    rpw = R // NW
    rb = RB if C <= 256 else RB // 2
    if (rpw // rb) % 2:
        rb //= 2
    nsteps = rpw // rb
    Cv = C // 16
    mesh = plsc.VectorSubcoreMesh(
        core_axis_name="c", subcore_axis_name="s",
        num_cores=NC, num_subcores=NS)

    scratch = [
        pltpu.VMEM((rpw * 8,), jnp.int32),
        pltpu.VMEM((rpw * 8,), F32),
        pltpu.VMEM((2, rb * 8, C), F32),
        pltpu.VMEM((2, rb, C), F32),
        pltpu.VMEM((2, rb, C), F32),
        pltpu.SemaphoreType.DMA((2,)),
        pltpu.SemaphoreType.DMA((2,)),
        pltpu.SemaphoreType.DMA((2,)),
    ]

    @functools.partial(
        pl.kernel,
        out_type=jax.ShapeDtypeStruct((R, C), F32),
        mesh=mesh,
        scratch_types=scratch,
    )
    def spmm(x_hbm, idx_hbm, w_hbm, *rest):
        if second:
            x0_hbm, out_hbm = rest[0], rest[1]
        else:
            out_hbm = rest[0]
            x0_hbm = None
        idx_v, w_v, rows, outb, x0b, sg, so, sx = rest[-8:]
        wid = lax.axis_index("s") * NC + lax.axis_index("c")
        base = wid * rpw
        pltpu.sync_copy(idx_hbm.at[pl.ds(base * 8, rpw * 8)], idx_v)
        pltpu.sync_copy(w_hbm.at[pl.ds(base * 8, rpw * 8)], w_v)

        def issue(s, p):
            pltpu.async_copy(
                x_hbm.at[idx_v.at[pl.ds(s * (rb * 8), rb * 8)]],
                rows.at[p], sg.at[p])
            if second:
                pltpu.async_copy(
                    x0_hbm.at[pl.ds(base + s * rb, rb)], x0b.at[p], sx.at[p])

        def compute(s, p):
            pltpu.make_async_copy(
                x_hbm.at[idx_v.at[pl.ds(s * (rb * 8), rb * 8)]],
                rows.at[p], sg.at[p]).wait()
            if second:
                pltpu.make_async_copy(
                    x0_hbm.at[pl.ds(base, rb)], x0b.at[p], sx.at[p]).wait()

            def rowpair(rr, carry2):
                wv = w_v[pl.ds(s * (rb * 8) + rr * 16, 16)]
                for half in range(2):
                    r = rr * 2 + half
                    accs = [jnp.zeros((16,), F32)] * Cv
                    for j in range(8):
                        lane = jnp.full((16,), half * 8 + j, jnp.int32)
                        wsp = wv.at[lane].get(mode="promise_in_bounds")
                        for c in range(Cv):
                            accs[c] = accs[c] + wsp * rows[
                                p, r * 8 + j, pl.ds(c * 16, 16)]
                    for c in range(Cv):
                        if second:
                            outb[p, r, pl.ds(c * 16, 16)] = (
                                2.0 * accs[c] - x0b[p, r, pl.ds(c * 16, 16)])
                        else:
                            outb[p, r, pl.ds(c * 16, 16)] = accs[c]
                return carry2

            lax.fori_loop(0, rb // 2, rowpair, 0, unroll=False)
            pltpu.async_copy(
                outb.at[p], out_hbm.at[pl.ds(base + s * rb, rb)], so.at[p])

        def drain_out(p):
            pltpu.make_async_copy(
                outb.at[p], out_hbm.at[pl.ds(base, rb)], so.at[p]).wait()

        issue(0, 0)
        issue(1, 1)

        def k_iter(k, carry):
            s0 = 2 * k

            @pl.when(k > 0)
            def _():
                drain_out(0)
            compute(s0, 0)

            @pl.when(s0 + 2 < nsteps)
            def _():
                issue(s0 + 2, 0)

            @pl.when(k > 0)
            def _():
                drain_out(1)
            compute(s0 + 1, 1)

            @pl.when(s0 + 3 < nsteps)
            def _():
                issue(s0 + 3, 1)
            return carry

        lax.fori_loop(0, nsteps // 2, k_iter, 0, unroll=False)
        drain_out(0)
        drain_out(1)

    return spmm


def _spmm(xf, idx, w):
    R, C = xf.shape
    return _make_spmm(R, C, False)(xf, idx, w)


def _spmm2(xf, idx, w, x0):
    R, C = xf.shape
    return _make_spmm(R, C, True)(xf, idx, w, x0)


@functools.lru_cache(maxsize=None)
def _make_combine(M, cins, cout, with_bias, with_stats, with_acc=False):
    n = len(cins)
    grid = (M // BM,)
    in_specs = [pl.BlockSpec((BM, cin), lambda i: (i, 0)) for cin in cins]
    in_specs += [pl.BlockSpec((cin, cout), lambda i: (0, 0)) for cin in cins]
    if with_acc:
        in_specs.append(pl.BlockSpec((BM, cout), lambda i: (i, 0)))
    if with_bias:
        in_specs.append(pl.BlockSpec((8, cout), lambda i: (0, 0)))
    out_shape = [jax.ShapeDtypeStruct((M, cout), F32)]
    out_specs = [pl.BlockSpec((BM, cout), lambda i: (i, 0))]
    if with_stats:
        out_shape.append(jax.ShapeDtypeStruct((8, cout), F32))
        out_specs.append(pl.BlockSpec((8, cout), lambda i: (0, 0)))

    def body(*refs):
        xr = refs[:n]
        wr = refs[n:2 * n]
        k = 2 * n
        ar = refs[k] if with_acc else None
        k += 1 if with_acc else 0
        br = refs[k] if with_bias else None
        k += 1 if with_bias else 0
        out_ref = refs[k]
        st_ref = refs[k + 1] if with_stats else None

        acc = jnp.dot(xr[0][...], wr[0][...], preferred_element_type=F32)
        for t in range(1, n):
            acc = acc + jnp.dot(xr[t][...], wr[t][...],
                                preferred_element_type=F32)
        if with_acc:
            acc = ar[...] + acc
        if with_bias:
            acc = acc + br[0, :][None, :]
        out_ref[...] = acc
        if with_stats:
            @pl.when(pl.program_id(0) == 0)
            def _():
                st_ref[...] = jnp.zeros_like(st_ref)
            st_ref[0, :] += jnp.sum(acc, axis=0)
            st_ref[1, :] += jnp.sum(acc * acc, axis=0)

    return pl.pallas_call(body, grid=grid, in_specs=in_specs,
                          out_specs=out_specs, out_shape=out_shape)


def _combine(xs, ws, bias=None, stats=False, acc=None):
    M = xs[0].shape[0]
    cins = tuple(x.shape[1] for x in xs)
    cout = ws[0].shape[1]
    args = list(xs) + list(ws)
    if acc is not None:
        args.append(acc)
    if bias is not None:
        args.append(bias)
    out = _make_combine(M, cins, cout, bias is not None, stats,
                        acc is not None)(*args)
    return out if stats else out[0]


@functools.lru_cache(maxsize=None)
def _make_bnrelu(M, C, with_z, Cp):
    grid = (M // BM,)
    in_specs = [pl.BlockSpec((BM, C), lambda i: (i, 0)),
                pl.BlockSpec((8, C), lambda i: (0, 0))]
    if with_z:
        in_specs.append(pl.BlockSpec((BM, C), lambda i: (i, 0)))

    def body(*refs):
        o_ref, p_ref = refs[0], refs[1]
        y_ref = refs[-1]
        y = jnp.maximum(o_ref[...] * p_ref[0, :][None, :]
                        + p_ref[1, :][None, :], 0.0)
        if with_z:
            y = y + refs[2][...] + p_ref[2, :][None, :]
        if Cp > C:
            y = jnp.concatenate([y, jnp.zeros((BM, Cp - C), F32)], axis=1)
        y_ref[...] = y

    return pl.pallas_call(
        body, grid=grid, in_specs=in_specs,
        out_specs=pl.BlockSpec((BM, Cp), lambda i: (i, 0)),
        out_shape=jax.ShapeDtypeStruct((M, Cp), F32))


def _bn_scale_shift(st, g, be, rows, resbias=None, packed=False):
    if packed:
        C = st.shape[1] // 2
        s0 = st[0, :C] + st[0, C:]
        s1 = st[1, :C] + st[1, C:]
        m = s0 / (2 * rows)
        v = s1 / (2 * rows) - m * m
        scale = g * lax.rsqrt(v + 1e-5)
        shift = be - m * scale
        scale = jnp.concatenate([scale, scale])
        shift = jnp.concatenate([shift, shift])
        if resbias is not None:
            resbias = jnp.concatenate([resbias, resbias])
    else:
        m = st[0] / rows
        v = st[1] / rows - m * m
        scale = g * lax.rsqrt(v + 1e-5)
        shift = be - m * scale
    p = jnp.zeros((8, st.shape[1]), F32).at[0].set(scale).at[1].set(shift)
    if resbias is not None:
        p = p.at[2].set(resbias)
    return p


def _blockdiag(W):
    ci, co = W.shape[-2], W.shape[-1]
    Z = jnp.zeros(W.shape[:-2] + (2 * ci, 2 * co), F32)
    return Z.at[..., :ci, :co].set(W).at[..., ci:, co:].set(W)


def _bnrelu(out, p, z=None, cpad=None):
    M, C = out.shape
    Cp = C if cpad is None else cpad
    if z is None:
        return _make_bnrelu(M, C, False, Cp)(out, p)
    return _make_bnrelu(M, C, True, Cp)(out, p, z)


@functools.lru_cache(maxsize=None)
def _make_pool(G, C):
    bg = min(BM, G)
    grid = (G // bg,)
    in_specs = [pl.BlockSpec((bg, C), lambda i, j=j: (i, j)) for j in range(4)]

    def body(a0, a1, a2, a3, m_ref, i_ref):
        x0, x1, x2, x3 = a0[...], a1[...], a2[...], a3[...]
        m = jnp.maximum(jnp.maximum(x0, x1), jnp.maximum(x2, x3))
        m_ref[...] = m
        i_ref[...] = jnp.where(
            x0 == m, 0,
            jnp.where(x1 == m, 1, jnp.where(x2 == m, 2, 3))).astype(jnp.int32)

    return pl.pallas_call(
        body, grid=grid, in_specs=in_specs,
        out_specs=[pl.BlockSpec((bg, C), lambda i: (i, 0)),
                   pl.BlockSpec((bg, C), lambda i: (i, 0))],
        out_shape=[jax.ShapeDtypeStruct((G, C), F32),
                   jax.ShapeDtypeStruct((G, C), jnp.int32)])


def _pool(xf):
    R, C = xf.shape
    xg = xf.reshape(R // 4, 4 * C)
    return _make_pool(R // 4, C)(xg, xg, xg, xg)


@functools.lru_cache(maxsize=None)
def _make_unpool(G, C):
    bg = min(BM, G)
    grid = (G // bg,)

    def body(x_ref, i_ref, o_ref):
        x = x_ref[...]
        idx = i_ref[...]
        o_ref[...] = jnp.concatenate(
            [jnp.where(idx == j, x, 0.0) for j in range(4)], axis=1)

    return pl.pallas_call(
        body, grid=grid,
        in_specs=[pl.BlockSpec((bg, C), lambda i: (i, 0)),
                  pl.BlockSpec((bg, C), lambda i: (i, 0))],
        out_specs=pl.BlockSpec((bg, 4 * C), lambda i: (i, 0)),
        out_shape=jax.ShapeDtypeStruct((G, 4 * C), F32))


def _unpool(xf, idx):
    G, C = xf.shape
    return _make_unpool(G, C)(xf, idx).reshape(G * 4, C)


def _cheb_split(xf, idx, w, W):
    x1 = _spmm(xf, idx, w)
    c01 = _combine([xf, x1], [W[0], W[1]])
    x2 = _spmm2(x1, idx, w, xf)
    return x2, W[2], c01


def _block(xf, idx, w, W, g, be, z=None, resbias=None, cpad=None,
           packed=False):
    x2, W2, c01 = _cheb_split(xf, idx, w, W)
    out, st = _combine([x2], [W2], stats=True, acc=c01)
    p = _bn_scale_shift(st, g, be, out.shape[0], resbias, packed)
    return _bnrelu(out, p, z, cpad)


def _pad_rows(W, rows):
    pad = [(0, 0)] * (W.ndim - 2) + [(0, rows - W.shape[-2]), (0, 0)]
    return jnp.pad(W, pad)


def kernel(x, src0, dst0, lw0, src1, dst1, lw1, src2, dst2, lw2,
           w_conv11, g_conv11, be_conv11, w_conv13, g_conv13, be_conv13,
           w_conv21, g_conv21, be_conv21, w_conv23, g_conv23, be_conv23,
           w_conv31, g_conv31, be_conv31, w_conv33, g_conv33, be_conv33,
           w_uconv21, g_uconv21, be_uconv21, w_uconv22, g_uconv22, be_uconv22,
           w_uconv11, g_uconv11, be_uconv11, w_uconv12, g_uconv12, be_uconv12,
           w_uconv13, b_uconv13,
           w_conv1res, b_conv1res, w_conv2res, b_conv2res,
           w_conv3res, b_conv3res):
    B, N0, C0 = x.shape
    N1, N2 = N0 // 4, N0 // 16
    src0 = src0.astype(jnp.int32)
    src1 = src1.astype(jnp.int32)
    src2 = src2.astype(jnp.int32)
    i0 = jnp.concatenate([src0, src0 + N0])
    e0 = jnp.concatenate([lw0, lw0])
    i1 = jnp.concatenate([src1, src1 + N1])
    e1 = jnp.concatenate([lw1, lw1])
    i2 = jnp.concatenate([src2, src2 + N2])
    e2 = jnp.concatenate([lw2, lw2])

    xf_pk = x.transpose(1, 0, 2).reshape(N0, 2 * C0)
    x11 = _block(xf_pk, src0, lw0, _blockdiag(w_conv11),
                 g_conv11, be_conv11, packed=True)
    res1 = _combine([xf_pk], [_blockdiag(w_conv1res)])
    x1 = _block(x11, src0, lw0, _blockdiag(w_conv13), g_conv13, be_conv13,
                z=res1, resbias=b_conv1res, packed=True)
    p1_pk, ix1_pk = _pool(x1)
    p1 = p1_pk.reshape(N1, 2, 128).transpose(1, 0, 2).reshape(2 * N1, 128)
    ix1 = ix1_pk.reshape(N1, 2, 128).transpose(1, 0, 2).reshape(2 * N1, 128)
    x2 = _block(p1, i1, e1, w_conv21, g_conv21, be_conv21, cpad=256)
    res2 = _combine([p1], [w_conv2res])
    x2 = _block(x2, i1, e1, _pad_rows(w_conv23, 256), g_conv23, be_conv23,
                z=res2, resbias=b_conv2res)
    p2, ix2 = _pool(x2)
    x3 = _block(p2, i2, e2, w_conv31, g_conv31, be_conv31)
    res3 = _combine([p2], [w_conv3res])
    x3 = _block(x3, i2, e2, w_conv33, g_conv33, be_conv33,
                z=res3, resbias=b_conv3res)
    u = _unpool(x3, ix2)
    u = jnp.concatenate([u, x2], axis=1)
    u = _block(u, i1, e1, w_uconv21, g_uconv21, be_uconv21)
    u = _block(u, i1, e1, w_uconv22, g_uconv22, be_uconv22)
    u = _unpool(u, ix1)
    x1u = x1.reshape(N0, 2, 128).transpose(1, 0, 2).reshape(2 * N0, 128)
    u = jnp.concatenate([u, x1u], axis=1)
    u = _block(u, i0, e0, w_uconv11, g_uconv11, be_uconv11)
    u = _block(u, i0, e0, w_uconv12, g_uconv12, be_uconv12)
    x11u = x11.reshape(N0, 2, 64).transpose(1, 0, 2).reshape(2 * N0, 64)
    u = jnp.concatenate([u, x11u], axis=1)
    cout = w_uconv13.shape[2]
    x2c, W2, c01 = _cheb_split(u, i0, e0, w_uconv13)
    bias = jnp.zeros((8, cout), F32).at[0].set(b_uconv13)
    out = _combine([x2c], [W2], bias=bias, acc=c01)
    return out.reshape(B, N0, cout)


# speedup vs baseline: 1.1670x; 1.0041x over previous
import functools

import jax
import jax.numpy as jnp
from jax import lax
from jax.experimental import pallas as pl
from jax.experimental.pallas import tpu as pltpu
from jax.experimental.pallas import tpu_sc as plsc

F32 = jnp.float32

NC = 2
NS = 16
NW = NC * NS
RB = 16
BM = 512


@functools.lru_cache(maxsize=None)
def _make_spmm(R, C, second):
    rpw = R // NW
    rb = RB if C <= 256 else RB // 2
    if (rpw // rb) % 2:
        rb //= 2
    nsteps = rpw // rb
    Cv = C // 16
    mesh = plsc.VectorSubcoreMesh(
        core_axis_name="c", subcore_axis_name="s",
        num_cores=NC, num_subcores=NS)

    scratch = [
        pltpu.VMEM((rpw * 8,), jnp.int32),
        pltpu.VMEM((rpw * 8,), F32),
        pltpu.VMEM((2, rb * 8, C), F32),
        pltpu.VMEM((2, rb, C), F32),
        pltpu.VMEM((2, rb, C), F32),
        pltpu.SemaphoreType.DMA((2,)),
        pltpu.SemaphoreType.DMA((2,)),
        pltpu.SemaphoreType.DMA((2,)),
    ]

    @functools.partial(
        pl.kernel,
        out_type=jax.ShapeDtypeStruct((R, C), F32),
        mesh=mesh,
        scratch_types=scratch,
    )
    def spmm(x_hbm, idx_hbm, w_hbm, *rest):
        if second:
            x0_hbm, out_hbm = rest[0], rest[1]
        else:
            out_hbm = rest[0]
            x0_hbm = None
        idx_v, w_v, rows, outb, x0b, sg, so, sx = rest[-8:]
        wid = lax.axis_index("s") * NC + lax.axis_index("c")
        base = wid * rpw
        pltpu.sync_copy(idx_hbm.at[pl.ds(base * 8, rpw * 8)], idx_v)
        pltpu.sync_copy(w_hbm.at[pl.ds(base * 8, rpw * 8)], w_v)

        def issue(s, p):
            pltpu.async_copy(
                x_hbm.at[idx_v.at[pl.ds(s * (rb * 8), rb * 8)]],
                rows.at[p], sg.at[p])
            if second:
                pltpu.async_copy(
                    x0_hbm.at[pl.ds(base + s * rb, rb)], x0b.at[p], sx.at[p])

        def compute(s, p):
            pltpu.make_async_copy(
                x_hbm.at[idx_v.at[pl.ds(s * (rb * 8), rb * 8)]],
                rows.at[p], sg.at[p]).wait()
            if second:
                pltpu.make_async_copy(
                    x0_hbm.at[pl.ds(base, rb)], x0b.at[p], sx.at[p]).wait()

            def rowpair(rr, carry2):
                wv = w_v[pl.ds(s * (rb * 8) + rr * 16, 16)]
                for half in range(2):
                    r = rr * 2 + half
                    accs = [jnp.zeros((16,), F32)] * Cv
                    for j in range(8):
                        lane = jnp.full((16,), half * 8 + j, jnp.int32)
                        wsp = wv.at[lane].get(mode="promise_in_bounds")
                        for c in range(Cv):
                            accs[c] = accs[c] + wsp * rows[
                                p, r * 8 + j, pl.ds(c * 16, 16)]
                    for c in range(Cv):
                        if second:
                            outb[p, r, pl.ds(c * 16, 16)] = (
                                2.0 * accs[c] - x0b[p, r, pl.ds(c * 16, 16)])
                        else:
                            outb[p, r, pl.ds(c * 16, 16)] = accs[c]
                return carry2

            lax.fori_loop(0, rb // 2, rowpair, 0, unroll=False)
            pltpu.async_copy(
                outb.at[p], out_hbm.at[pl.ds(base + s * rb, rb)], so.at[p])

        def drain_out(p):
            pltpu.make_async_copy(
                outb.at[p], out_hbm.at[pl.ds(base, rb)], so.at[p]).wait()

        issue(0, 0)
        issue(1, 1)

        def k_iter(k, carry):
            s0 = 2 * k

            @pl.when(k > 0)
            def _():
                drain_out(0)
            compute(s0, 0)

            @pl.when(s0 + 2 < nsteps)
            def _():
                issue(s0 + 2, 0)

            @pl.when(k > 0)
            def _():
                drain_out(1)
            compute(s0 + 1, 1)

            @pl.when(s0 + 3 < nsteps)
            def _():
                issue(s0 + 3, 1)
            return carry

        lax.fori_loop(0, nsteps // 2, k_iter, 0, unroll=False)
        drain_out(0)
        drain_out(1)

    return spmm


def _spmm(xf, idx, w):
    R, C = xf.shape
    return _make_spmm(R, C, False)(xf, idx, w)


def _spmm2(xf, idx, w, x0):
    R, C = xf.shape
    return _make_spmm(R, C, True)(xf, idx, w, x0)


@functools.lru_cache(maxsize=None)
def _make_combine(M, cins, cout, with_bias, with_stats, with_acc=False):
    n = len(cins)
    grid = (M // BM,)
    in_specs = [pl.BlockSpec((BM, cin), lambda i: (i, 0)) for cin in cins]
    in_specs += [pl.BlockSpec((cin, cout), lambda i: (0, 0)) for cin in cins]
    if with_acc:
        in_specs.append(pl.BlockSpec((BM, cout), lambda i: (i, 0)))
    if with_bias:
        in_specs.append(pl.BlockSpec((8, cout), lambda i: (0, 0)))
    out_shape = [jax.ShapeDtypeStruct((M, cout), F32)]
    out_specs = [pl.BlockSpec((BM, cout), lambda i: (i, 0))]
    if with_stats:
        out_shape.append(jax.ShapeDtypeStruct((8, cout), F32))
        out_specs.append(pl.BlockSpec((8, cout), lambda i: (0, 0)))

    def body(*refs):
        xr = refs[:n]
        wr = refs[n:2 * n]
        k = 2 * n
        ar = refs[k] if with_acc else None
        k += 1 if with_acc else 0
        br = refs[k] if with_bias else None
        k += 1 if with_bias else 0
        out_ref = refs[k]
        st_ref = refs[k + 1] if with_stats else None

        acc = jnp.dot(xr[0][...], wr[0][...], preferred_element_type=F32)
        for t in range(1, n):
            acc = acc + jnp.dot(xr[t][...], wr[t][...],
                                preferred_element_type=F32)
        if with_acc:
            acc = ar[...] + acc
        if with_bias:
            acc = acc + br[0, :][None, :]
        out_ref[...] = acc
        if with_stats:
            @pl.when(pl.program_id(0) == 0)
            def _():
                st_ref[...] = jnp.zeros_like(st_ref)
            st_ref[0, :] += jnp.sum(acc, axis=0)
            st_ref[1, :] += jnp.sum(acc * acc, axis=0)

    return pl.pallas_call(body, grid=grid, in_specs=in_specs,
                          out_specs=out_specs, out_shape=out_shape)


def _combine(xs, ws, bias=None, stats=False, acc=None):
    M = xs[0].shape[0]
    cins = tuple(x.shape[1] for x in xs)
    cout = ws[0].shape[1]
    args = list(xs) + list(ws)
    if acc is not None:
        args.append(acc)
    if bias is not None:
        args.append(bias)
    out = _make_combine(M, cins, cout, bias is not None, stats,
                        acc is not None)(*args)
    return out if stats else out[0]


@functools.lru_cache(maxsize=None)
def _make_bnrelu(M, C, with_z, Cp):
    grid = (M // BM,)
    in_specs = [pl.BlockSpec((BM, C), lambda i: (i, 0)),
                pl.BlockSpec((8, C), lambda i: (0, 0))]
    if with_z:
        in_specs.append(pl.BlockSpec((BM, C), lambda i: (i, 0)))

    def body(*refs):
        o_ref, p_ref = refs[0], refs[1]
        y_ref = refs[-1]
        y = jnp.maximum(o_ref[...] * p_ref[0, :][None, :]
                        + p_ref[1, :][None, :], 0.0)
        if with_z:
            y = y + refs[2][...] + p_ref[2, :][None, :]
        if Cp > C:
            y = jnp.concatenate([y, jnp.zeros((BM, Cp - C), F32)], axis=1)
        y_ref[...] = y

    return pl.pallas_call(
        body, grid=grid, in_specs=in_specs,
        out_specs=pl.BlockSpec((BM, Cp), lambda i: (i, 0)),
        out_shape=jax.ShapeDtypeStruct((M, Cp), F32))


def _bn_scale_shift(st, g, be, rows, resbias=None, packed=False):
    if packed:
        C = st.shape[1] // 2
        s0 = st[0, :C] + st[0, C:]
        s1 = st[1, :C] + st[1, C:]
        m = s0 / (2 * rows)
        v = s1 / (2 * rows) - m * m
        scale = g * lax.rsqrt(v + 1e-5)
        shift = be - m * scale
        scale = jnp.concatenate([scale, scale])
        shift = jnp.concatenate([shift, shift])
        if resbias is not None:
            resbias = jnp.concatenate([resbias, resbias])
    else:
        m = st[0] / rows
        v = st[1] / rows - m * m
        scale = g * lax.rsqrt(v + 1e-5)
        shift = be - m * scale
    p = jnp.zeros((8, st.shape[1]), F32).at[0].set(scale).at[1].set(shift)
    if resbias is not None:
        p = p.at[2].set(resbias)
    return p


def _blockdiag(W):
    ci, co = W.shape[-2], W.shape[-1]
    Z = jnp.zeros(W.shape[:-2] + (2 * ci, 2 * co), F32)
    return Z.at[..., :ci, :co].set(W).at[..., ci:, co:].set(W)


def _bnrelu(out, p, z=None, cpad=None):
    M, C = out.shape
    Cp = C if cpad is None else cpad
    if z is None:
        return _make_bnrelu(M, C, False, Cp)(out, p)
    return _make_bnrelu(M, C, True, Cp)(out, p, z)


@functools.lru_cache(maxsize=None)
def _make_pool(G, C):
    bg = min(BM, G)
    grid = (G // bg,)
    in_specs = [pl.BlockSpec((bg, C), lambda i, j=j: (i, j)) for j in range(4)]

    def body(a0, a1, a2, a3, m_ref, i_ref):
        x0, x1, x2, x3 = a0[...], a1[...], a2[...], a3[...]
        m = jnp.maximum(jnp.maximum(x0, x1), jnp.maximum(x2, x3))
        m_ref[...] = m
        i_ref[...] = jnp.where(
            x0 == m, 0,
            jnp.where(x1 == m, 1, jnp.where(x2 == m, 2, 3))).astype(jnp.int32)

    return pl.pallas_call(
        body, grid=grid, in_specs=in_specs,
        out_specs=[pl.BlockSpec((bg, C), lambda i: (i, 0)),
                   pl.BlockSpec((bg, C), lambda i: (i, 0))],
        out_shape=[jax.ShapeDtypeStruct((G, C), F32),
                   jax.ShapeDtypeStruct((G, C), jnp.int32)])


def _pool(xf):
    R, C = xf.shape
    xg = xf.reshape(R // 4, 4 * C)
    return _make_pool(R // 4, C)(xg, xg, xg, xg)


@functools.lru_cache(maxsize=None)
def _make_unpool(G, C):
    bg = min(BM, G)
    grid = (G // bg,)

    def body(x_ref, i_ref, o_ref):
        x = x_ref[...]
        idx = i_ref[...]
        o_ref[...] = jnp.concatenate(
            [jnp.where(idx == j, x, 0.0) for j in range(4)], axis=1)

    return pl.pallas_call(
        body, grid=grid,
        in_specs=[pl.BlockSpec((bg, C), lambda i: (i, 0)),
                  pl.BlockSpec((bg, C), lambda i: (i, 0))],
        out_specs=pl.BlockSpec((bg, 4 * C), lambda i: (i, 0)),
        out_shape=jax.ShapeDtypeStruct((G, 4 * C), F32))


def _unpool(xf, idx):
    G, C = xf.shape
    return _make_unpool(G, C)(xf, idx).reshape(G * 4, C)


def _cheb_split(xf, idx, w, W, split=True):
    x1 = _spmm(xf, idx, w)
    c01 = _combine([xf, x1], [W[0], W[1]]) if split else None
    x2 = _spmm2(x1, idx, w, xf)
    if split:
        return [x2], [W[2]], c01
    return [xf, x1, x2], [W[0], W[1], W[2]], None


SPLIT = False


def _block(xf, idx, w, W, g, be, z=None, resbias=None, cpad=None,
           packed=False):
    xs, ws, c01 = _cheb_split(xf, idx, w, W, SPLIT)
    out, st = _combine(xs, ws, stats=True, acc=c01)
    p = _bn_scale_shift(st, g, be, out.shape[0], resbias, packed)
    return _bnrelu(out, p, z, cpad)


def _pad_rows(W, rows):
    pad = [(0, 0)] * (W.ndim - 2) + [(0, rows - W.shape[-2]), (0, 0)]
    return jnp.pad(W, pad)


def kernel(x, src0, dst0, lw0, src1, dst1, lw1, src2, dst2, lw2,
           w_conv11, g_conv11, be_conv11, w_conv13, g_conv13, be_conv13,
           w_conv21, g_conv21, be_conv21, w_conv23, g_conv23, be_conv23,
           w_conv31, g_conv31, be_conv31, w_conv33, g_conv33, be_conv33,
           w_uconv21, g_uconv21, be_uconv21, w_uconv22, g_uconv22, be_uconv22,
           w_uconv11, g_uconv11, be_uconv11, w_uconv12, g_uconv12, be_uconv12,
           w_uconv13, b_uconv13,
           w_conv1res, b_conv1res, w_conv2res, b_conv2res,
           w_conv3res, b_conv3res):
    B, N0, C0 = x.shape
    N1, N2 = N0 // 4, N0 // 16
    src0 = src0.astype(jnp.int32)
    src1 = src1.astype(jnp.int32)
    src2 = src2.astype(jnp.int32)
    i0 = jnp.concatenate([src0, src0 + N0])
    e0 = jnp.concatenate([lw0, lw0])
    i1 = jnp.concatenate([src1, src1 + N1])
    e1 = jnp.concatenate([lw1, lw1])
    i2 = jnp.concatenate([src2, src2 + N2])
    e2 = jnp.concatenate([lw2, lw2])

    xf_pk = x.transpose(1, 0, 2).reshape(N0, 2 * C0)
    x11 = _block(xf_pk, src0, lw0, _blockdiag(w_conv11),
                 g_conv11, be_conv11, packed=True)
    res1 = _combine([xf_pk], [_blockdiag(w_conv1res)])
    x1 = _block(x11, src0, lw0, _blockdiag(w_conv13), g_conv13, be_conv13,
                z=res1, resbias=b_conv1res, packed=True)
    p1_pk, ix1_pk = _pool(x1)
    p1 = p1_pk.reshape(N1, 2, 128).transpose(1, 0, 2).reshape(2 * N1, 128)
    ix1 = ix1_pk.reshape(N1, 2, 128).transpose(1, 0, 2).reshape(2 * N1, 128)
    x2 = _block(p1, i1, e1, w_conv21, g_conv21, be_conv21, cpad=256)
    res2 = _combine([p1], [w_conv2res])
    x2 = _block(x2, i1, e1, _pad_rows(w_conv23, 256), g_conv23, be_conv23,
                z=res2, resbias=b_conv2res)
    p2, ix2 = _pool(x2)
    x3 = _block(p2, i2, e2, w_conv31, g_conv31, be_conv31)
    res3 = _combine([p2], [w_conv3res])
    x3 = _block(x3, i2, e2, w_conv33, g_conv33, be_conv33,
                z=res3, resbias=b_conv3res)
    u = _unpool(x3, ix2)
    u = jnp.concatenate([u, x2], axis=1)
    u = _block(u, i1, e1, w_uconv21, g_uconv21, be_uconv21)
    u = _block(u, i1, e1, w_uconv22, g_uconv22, be_uconv22)
    u = _unpool(u, ix1)
    x1u = x1.reshape(N0, 2, 128).transpose(1, 0, 2).reshape(2 * N0, 128)
    u = jnp.concatenate([u, x1u], axis=1)
    u = _block(u, i0, e0, w_uconv11, g_uconv11, be_uconv11)
    u = _block(u, i0, e0, w_uconv12, g_uconv12, be_uconv12)
    x11u = x11.reshape(N0, 2, 64).transpose(1, 0, 2).reshape(2 * N0, 64)
    u = jnp.concatenate([u, x11u], axis=1)
    cout = w_uconv13.shape[2]
    xsf, wsf, c01 = _cheb_split(u, i0, e0, w_uconv13, SPLIT)
    bias = jnp.zeros((8, cout), F32).at[0].set(b_uconv13)
    out = _combine(xsf, wsf, bias=bias, acc=c01)
    return out.reshape(B, N0, cout)
